# Initial kernel scaffold; baseline (speedup 1.0000x reference)
import jax
import jax.numpy as jnp
from jax.experimental import pallas as pl


def kernel(node_feats, edge_feats, edge_indices, adj, W0, a_src0, a_dst0, a_e0, W1, a_src1, a_dst1, a_e1):
    raise NotImplementedError("write your pallas kernel here")


# speedup vs baseline: 2.2696x; 2.2696x over previous
import functools

import jax
import jax.numpy as jnp
from jax import lax
from jax.experimental import pallas as pl
from jax.experimental.pallas import tpu as pltpu
from jax.experimental.pallas import tpu_sc as plsc

NEG = 0.2
AUG = 48
NC, NS, LANES = 2, 16, 16
CHUNK = 128
BROW = 80


def _prep_body(x_ref, w_ref, asrc_ref, adst_ref, h_ref, s_ref, d_ref, g_ref,
               *, hdim):
    h = jnp.dot(x_ref[...], w_ref[...], preferred_element_type=jnp.float32)
    h_ref[...] = h
    s_ref[...] = jnp.dot(h, asrc_ref[...], preferred_element_type=jnp.float32)
    d_ref[...] = jnp.dot(h, adst_ref[...], preferred_element_type=jnp.float32)
    n = h.shape[0]
    pad = jnp.zeros((n, AUG - 1 - hdim), jnp.float32)
    ones = jnp.ones((n, 1), jnp.float32)
    g_ref[...] = jnp.concatenate([h, pad, ones], axis=1)


def _prep(x, w, asrc, adst):
    n, hdim = x.shape[0], w.shape[1]
    return pl.pallas_call(
        functools.partial(_prep_body, hdim=hdim),
        out_shape=[
            jax.ShapeDtypeStruct((n, hdim), jnp.float32),
            jax.ShapeDtypeStruct((n, 1), jnp.float32),
            jax.ShapeDtypeStruct((n, 1), jnp.float32),
            jax.ShapeDtypeStruct((n, AUG), jnp.float32),
        ],
    )(x, w, asrc.reshape(-1, 1), adst.reshape(-1, 1))


def _dense_body(adj_ref, s_ref, d_ref, h_ref, dacc_ref, o_ref, *, hdim, elu):
    t = s_ref[...] + d_ref[...]
    p = jnp.exp(jnp.where(t > 0, t, NEG * t))
    p = jnp.where(adj_ref[...] > 0, p, 0.0)
    bsum = jnp.sum(p, axis=1, keepdims=True)
    bvec = jnp.dot(p, h_ref[...], preferred_element_type=jnp.float32)
    dacc = dacc_ref[...]
    dvec = dacc[0, :, :hdim] + dacc[1, :, :hdim]
    dsum = dacc[0, :, AUG - 1:AUG] + dacc[1, :, AUG - 1:AUG]
    x = (bvec + dvec) / (bsum + dsum)
    if elu:
        x = jnp.where(x > 0, x, jnp.exp(x) - 1.0)
    o_ref[...] = x


def _dense(adj, s, drow, h, dacc, *, elu):
    n, hdim = h.shape
    grid = (n // BROW,)
    return pl.pallas_call(
        functools.partial(_dense_body, hdim=hdim, elu=elu),
        grid=grid,
        in_specs=[
            pl.BlockSpec((BROW, n), lambda i: (i, 0)),
            pl.BlockSpec((BROW, 1), lambda i: (i, 0)),
            pl.BlockSpec((1, n), lambda i: (0, 0)),
            pl.BlockSpec((n, hdim), lambda i: (0, 0)),
            pl.BlockSpec((2, BROW, AUG), lambda i: (0, i, 0)),
        ],
        out_specs=pl.BlockSpec((BROW, hdim), lambda i: (i, 0)),
        out_shape=jax.ShapeDtypeStruct((n, hdim), jnp.float32),
        compiler_params=pltpu.CompilerParams(
            dimension_semantics=("arbitrary",)),
    )(adj, s, drow, h, dacc)


def _edge_body(n, epw, nchunks,
               i_hbm, j_hbm, ef_hbm, ae_hbm, s_hbm, d_hbm, adj_hbm, g_hbm,
               zero_hbm, dacc_hbm,
               s_v, d_v, ae_v, iv, jv, efv, aidx, adjv, growsv, scaledv,
               deltav, acc, sem_l, sem_a, sem_g, sem_s):
    core = lax.axis_index("c")
    sid = lax.axis_index("s")
    wid = sid * NC + core

    @pl.when(sid == 0)
    def _():
        pltpu.sync_copy(zero_hbm, acc)
    plsc.subcore_barrier()

    pltpu.sync_copy(s_hbm, s_v)
    pltpu.sync_copy(d_hbm, d_v)
    pltpu.sync_copy(ae_hbm, ae_v)

    lane = lax.iota(jnp.int32, 16)
    ngrp = CHUNK // LANES

    @pl.loop(0, nchunks)
    def _chunk(c):
        base = wid * epw + c * CHUNK
        di = pltpu.async_copy(i_hbm.at[pl.ds(base, CHUNK)], iv, sem_l)
        dj = pltpu.async_copy(j_hbm.at[pl.ds(base, CHUNK)], jv, sem_l)
        de = pltpu.async_copy(ef_hbm.at[pl.ds(base * 4, CHUNK * 4)], efv,
                              sem_l)
        dj.wait()
        dg = pltpu.async_copy(g_hbm.at[jv], growsv, sem_g)
        di.wait()
        for g in range(ngrp):
            i16 = iv[pl.ds(g * 16, 16)]
            j16 = jv[pl.ds(g * 16, 16)]
            aidx[pl.ds(g * 16, 16)] = i16 * n + j16
        da = pltpu.async_copy(adj_hbm.at[aidx], adjv, sem_a)
        de.wait()
        da.wait()
        aev = ae_v[...]
        ae0 = aev[0]
        ae1 = aev[1]
        ae2 = aev[2]
        ae3 = aev[3]
        for g in range(ngrp):
            i16 = iv[pl.ds(g * 16, 16)]
            j16 = jv[pl.ds(g * 16, 16)]
            s16 = plsc.load_gather(s_v, [i16])
            d16 = plsc.load_gather(d_v, [j16])
            eidx = (lane + g * 16) * 4
            ee16 = (plsc.load_gather(efv, [eidx]) * ae0 +
                    plsc.load_gather(efv, [eidx + 1]) * ae1 +
                    plsc.load_gather(efv, [eidx + 2]) * ae2 +
                    plsc.load_gather(efv, [eidx + 3]) * ae3)
            a16 = adjv[pl.ds(g * 16, 16)]
            t = s16 + d16
            g1 = jnp.exp(jnp.where(t > 0, t, NEG * t))
            t2 = t + ee16
            g2 = jnp.exp(jnp.where(t2 > 0, t2, NEG * t2))
            deltav[pl.ds(g * 16, 16)] = jnp.where(a16 > 0, g2 - g1, 0.0)
        dg.wait()
        for g in range(ngrp):
            d16 = deltav[pl.ds(g * 16, 16)]
            for l in range(16):
                ei = g * 16 + l
                dlt = d16[l]
                for k in range(AUG // 16):
                    scaledv[ei, pl.ds(k * 16, 16)] = (
                        growsv[ei, pl.ds(k * 16, 16)] * dlt)
        pltpu.async_copy(scaledv, acc.at[iv], sem_s, add=True).wait()

    plsc.subcore_barrier()

    @pl.when(sid == 0)
    def _():
        pltpu.sync_copy(acc, dacc_hbm.at[core])


def _edges(ipad, jpad, efflat, ae, s, d, adjflat, gtab, zeros_acc):
    n = s.shape[0]
    e_pad = ipad.shape[0]
    epw = e_pad // (NC * NS)
    nchunks = epw // CHUNK
    mesh = plsc.VectorSubcoreMesh(core_axis_name="c", subcore_axis_name="s")
    body = functools.partial(_edge_body, n, epw, nchunks)
    return pl.kernel(
        body,
        out_type=jax.ShapeDtypeStruct((NC, n, AUG), jnp.float32),
        mesh=mesh,
        compiler_params=pltpu.CompilerParams(
            needs_layout_passes=False, use_tc_tiling_on_sc=False),
        scratch_types=[
            pltpu.VMEM((n,), jnp.float32),
            pltpu.VMEM((n,), jnp.float32),
            pltpu.VMEM((16,), jnp.float32),
            pltpu.VMEM((CHUNK,), jnp.int32),
            pltpu.VMEM((CHUNK,), jnp.int32),
            pltpu.VMEM((CHUNK * 4,), jnp.float32),
            pltpu.VMEM((CHUNK,), jnp.int32),
            pltpu.VMEM((CHUNK,), jnp.int32),
            pltpu.VMEM((CHUNK, AUG), jnp.float32),
            pltpu.VMEM((CHUNK, AUG), jnp.float32),
            pltpu.VMEM((CHUNK,), jnp.float32),
            pltpu.VMEM_SHARED((n, AUG), jnp.float32),
            pltpu.SemaphoreType.DMA,
            pltpu.SemaphoreType.DMA,
            pltpu.SemaphoreType.DMA,
            pltpu.SemaphoreType.DMA,
        ],
    )(ipad, jpad, efflat, ae, s, d, adjflat, gtab, zeros_acc)


def kernel(node_feats, edge_feats, edge_indices, adj, W0, a_src0, a_dst0,
           a_e0, W1, a_src1, a_dst1, a_e1):
    n = node_feats.shape[0]
    e = edge_feats.shape[0]
    quant = NC * NS * CHUNK
    e_pad = ((e + quant - 1) // quant) * quant

    ipad = jnp.concatenate(
        [edge_indices[0], jnp.zeros((e_pad - e,), edge_indices.dtype)])
    jpad = jnp.concatenate(
        [edge_indices[1], jnp.zeros((e_pad - e,), edge_indices.dtype)])
    efflat = jnp.concatenate(
        [edge_feats, jnp.zeros((e_pad - e, edge_feats.shape[1]),
                               edge_feats.dtype)]).reshape(-1)
    adjflat = adj.reshape(n * n)
    zeros_acc = jnp.zeros((n, AUG), jnp.float32)
    ae0 = jnp.concatenate([a_e0, jnp.zeros((12,), jnp.float32)])
    ae1 = jnp.concatenate([a_e1, jnp.zeros((12,), jnp.float32)])

    h0, s0, d0, g0 = _prep(node_feats, W0, a_src0, a_dst0)
    dacc0 = _edges(ipad, jpad, efflat, ae0, s0.reshape(-1), d0.reshape(-1),
                   adjflat, g0, zeros_acc)
    x1 = _dense(adj, s0, d0.reshape(1, -1), h0, dacc0, elu=False)

    h1, s1, d1, g1 = _prep(x1, W1, a_src1, a_dst1)
    dacc1 = _edges(ipad, jpad, efflat, ae1, s1.reshape(-1), d1.reshape(-1),
                   adjflat, g1, zeros_acc)
    out = _dense(adj, s1, d1.reshape(1, -1), h1, dacc1, elu=True)
    return out
